# Initial kernel scaffold; baseline (speedup 1.0000x reference)
import functools

import jax
import jax.numpy as jnp
from jax import lax
from jax.experimental import pallas as pl
from jax.experimental.pallas import tpu as pltpu
from jax.experimental.pallas import tpu_sc as plsc

N = 10000
D = 128
G = 64

NTILES = 32
CH = 128
CHUNKS = 79
EPT = CH * CHUNKS
EP = NTILES * EPT
NACC = 10240
ZROWS = 640


_sc_mesh = plsc.VectorSubcoreMesh(core_axis_name="c", subcore_axis_name="s")


@functools.partial(
    pl.kernel,
    out_type=jax.ShapeDtypeStruct((2, N, D), jnp.float32),
    scratch_types=[
        pltpu.VMEM((CHUNKS, CH), jnp.int32),
        pltpu.VMEM((CHUNKS, CH), jnp.int32),
        pltpu.VMEM((CH, D), jnp.float32),
        pltpu.VMEM_SHARED((NACC, D), jnp.float32),
        pltpu.SemaphoreType.DMA,
    ],
    mesh=_sc_mesh,
)
def _sc_aggregate(x_hbm, srcp_hbm, dstp_hbm, zeros_hbm, out_hbm,
                  src_v, dst_v, rows_v, acc, sem):
    c = lax.axis_index("c")
    s = lax.axis_index("s")
    tid = c * 16 + s

    pltpu.sync_copy(srcp_hbm.at[pl.ds(tid * CHUNKS, CHUNKS)], src_v)
    pltpu.sync_copy(dstp_hbm.at[pl.ds(tid * CHUNKS, CHUNKS)], dst_v)

    pltpu.sync_copy(zeros_hbm, acc.at[pl.ds(s * ZROWS, ZROWS)])
    plsc.subcore_barrier()

    def chunk_body(ci, carry):
        pltpu.async_copy(x_hbm.at[src_v.at[ci]], rows_v, sem).wait()
        pltpu.sync_copy(rows_v, acc.at[dst_v.at[ci]], add=True)
        return carry

    lax.fori_loop(0, CHUNKS, chunk_body, 0)
    plsc.subcore_barrier()

    pltpu.sync_copy(acc.at[pl.ds(s * 625, 625)], out_hbm.at[c, pl.ds(s * 625, 625)])


ROWS1 = 1000


def _mlp1_body(eps_ref, x_ref, p0_ref, p1_ref, w1_ref, b1_ref, h_ref, st_ref):
    i = pl.program_id(0)
    a = x_ref[...] * (1.0 + eps_ref[0, 0]) + p0_ref[...] + p1_ref[...]
    h = jnp.dot(a, w1_ref[...], preferred_element_type=jnp.float32) + b1_ref[...]
    h_ref[...] = h
    s = jnp.sum(h, axis=0, keepdims=True)
    s2 = jnp.sum(h * h, axis=0, keepdims=True)

    @pl.when(i == 0)
    def _():
        st_ref[0:1, :] = s
        st_ref[1:2, :] = s2

    @pl.when(i > 0)
    def _():
        st_ref[0:1, :] += s
        st_ref[1:2, :] += s2


_mlp1 = pl.pallas_call(
    _mlp1_body,
    grid=(N // ROWS1,),
    in_specs=[
        pl.BlockSpec(memory_space=pltpu.SMEM),
        pl.BlockSpec((ROWS1, D), lambda i: (i, 0)),
        pl.BlockSpec((ROWS1, D), lambda i: (i, 0)),
        pl.BlockSpec((ROWS1, D), lambda i: (i, 0)),
        pl.BlockSpec((D, D), lambda i: (0, 0)),
        pl.BlockSpec((1, D), lambda i: (0, 0)),
    ],
    out_specs=[
        pl.BlockSpec((ROWS1, D), lambda i: (i, 0)),
        pl.BlockSpec((2, D), lambda i: (0, 0)),
    ],
    out_shape=[
        jax.ShapeDtypeStruct((N, D), jnp.float32),
        jax.ShapeDtypeStruct((2, D), jnp.float32),
    ],
    compiler_params=pltpu.CompilerParams(dimension_semantics=("arbitrary",)),
)


def _mlp2_body(h_ref, st_ref, bng_ref, bnb_ref, w2_ref, b2_ref, lng_ref,
               lnb_ref, out_ref):
    m = st_ref[0:1, :] / N
    v = st_ref[1:2, :] / N - m * m
    h = (h_ref[...] - m) * lax.rsqrt(v + 1e-5) * bng_ref[...] + bnb_ref[...]
    h = jnp.maximum(h, 0.0)
    h = jnp.dot(h, w2_ref[...], preferred_element_type=jnp.float32) + b2_ref[...]
    mu = jnp.mean(h, axis=1, keepdims=True)
    va = jnp.mean(h * h, axis=1, keepdims=True) - mu * mu
    h = (h - mu) * lax.rsqrt(va + 1e-5) * lng_ref[...] + lnb_ref[...]
    out_ref[...] = jnp.where(h > 0, h, 0.1 * h)


_mlp2 = pl.pallas_call(
    _mlp2_body,
    grid=(N // ROWS1,),
    in_specs=[
        pl.BlockSpec((ROWS1, D), lambda i: (i, 0)),
        pl.BlockSpec((2, D), lambda i: (0, 0)),
        pl.BlockSpec((1, D), lambda i: (0, 0)),
        pl.BlockSpec((1, D), lambda i: (0, 0)),
        pl.BlockSpec((D, D), lambda i: (0, 0)),
        pl.BlockSpec((1, D), lambda i: (0, 0)),
        pl.BlockSpec((1, D), lambda i: (0, 0)),
        pl.BlockSpec((1, D), lambda i: (0, 0)),
    ],
    out_specs=pl.BlockSpec((ROWS1, D), lambda i: (i, 0)),
    out_shape=jax.ShapeDtypeStruct((N, D), jnp.float32),
    compiler_params=pltpu.CompilerParams(dimension_semantics=("arbitrary",)),
)


ROWSP = 200


def _pool_body(x_ref, b_ref, wf_ref, bf_ref, out_ref, mx, sm, cnt):
    i = pl.program_id(0)

    @pl.when(i == 0)
    def _():
        mx[...] = jnp.full((G, D), -jnp.inf, jnp.float32)
        sm[...] = jnp.zeros((G, D), jnp.float32)
        cnt[...] = jnp.zeros((G, 1), jnp.float32)

    x = x_ref[...]
    gids = lax.broadcasted_iota(jnp.int32, (1, G), 1)
    oh = (b_ref[...] == gids).astype(jnp.float32)
    dn = (((0,), (0,)), ((), ()))
    sm[...] += lax.dot_general(oh, x, dn, preferred_element_type=jnp.float32)
    cnt[...] += lax.dot_general(oh, jnp.ones((ROWSP, 1), jnp.float32), dn,
                                preferred_element_type=jnp.float32)
    big = jnp.where((oh > 0)[:, :, None], x[:, None, :], -jnp.inf)
    mx[...] = jnp.maximum(mx[...], jnp.max(big, axis=0))

    @pl.when(i == pl.num_programs(0) - 1)
    def _():
        mean = sm[...] / jnp.maximum(cnt[...], 1.0)
        res = jnp.dot(mx[...], wf_ref[0:D, :], preferred_element_type=jnp.float32)
        res += jnp.dot(mean, wf_ref[D:2 * D, :], preferred_element_type=jnp.float32)
        out_ref[...] = res + bf_ref[...]


_pool = pl.pallas_call(
    _pool_body,
    grid=(N // ROWSP,),
    in_specs=[
        pl.BlockSpec((ROWSP, D), lambda i: (i, 0)),
        pl.BlockSpec((ROWSP, 1), lambda i: (i, 0)),
        pl.BlockSpec((2 * D, D), lambda i: (0, 0)),
        pl.BlockSpec((1, D), lambda i: (0, 0)),
    ],
    out_specs=pl.BlockSpec((G, D), lambda i: (0, 0)),
    out_shape=jax.ShapeDtypeStruct((G, D), jnp.float32),
    scratch_shapes=[
        pltpu.VMEM((G, D), jnp.float32),
        pltpu.VMEM((G, D), jnp.float32),
        pltpu.VMEM((G, 1), jnp.float32),
    ],
    compiler_params=pltpu.CompilerParams(dimension_semantics=("arbitrary",)),
)


def kernel(x, edge_index, batch, W1, b1, bn_g, bn_b, W2, b2, eps, ln_g, ln_b,
           Wf, bf):
    E = edge_index.shape[1]
    pad = EP - E
    src = jnp.concatenate([edge_index[0], jnp.zeros((pad,), jnp.int32)])
    dst = jnp.concatenate([edge_index[1], jnp.full((pad,), N, jnp.int32)])
    srcp = src.reshape(EP // CH, CH)
    dstp = dst.reshape(EP // CH, CH)
    zeros_blk = jnp.zeros((ZROWS, D), jnp.float32)
    batch2 = batch.reshape(N, 1)

    out = x
    for l in range(W1.shape[0]):
        partials = _sc_aggregate(out, srcp, dstp, zeros_blk)
        h1, st = _mlp1(eps[l].reshape(1, 1), out, partials[0], partials[1],
                       W1[l], b1[l].reshape(1, D))
        out = _mlp2(h1, st, bn_g[l].reshape(1, D), bn_b[l].reshape(1, D),
                    W2[l], b2[l].reshape(1, D), ln_g[l].reshape(1, D),
                    ln_b[l].reshape(1, D))
    return _pool(out, batch2, Wf, bf.reshape(1, D))


# speedup vs baseline: 2.7283x; 2.7283x over previous
import functools

import jax
import jax.numpy as jnp
from jax import lax
from jax.experimental import pallas as pl
from jax.experimental.pallas import tpu as pltpu
from jax.experimental.pallas import tpu_sc as plsc

N = 10000
D = 128
G = 64

NTILES = 32
CH = 128
CHUNKS = 80
EPT = CH * CHUNKS
EP = NTILES * EPT
NACC = 10240
ZROWS = 640


@functools.cache
def _get_sc_aggregate():
    mesh = plsc.VectorSubcoreMesh(core_axis_name="c", subcore_axis_name="s")

    @functools.partial(
        pl.kernel,
        out_type=jax.ShapeDtypeStruct((2, NACC, D), jnp.float32),
        scratch_types=[
            pltpu.VMEM((CHUNKS, CH), jnp.int32),
            pltpu.VMEM((CHUNKS, CH), jnp.int32),
            pltpu.VMEM((CH, D), jnp.float32),
            pltpu.VMEM_SHARED((NACC, D), jnp.float32),
            pltpu.SemaphoreType.DMA,
        ],
        mesh=mesh,
    )
    def _sc_aggregate(x_hbm, srcp_hbm, dstp_hbm, zeros_hbm, out_hbm,
                      src_v, dst_v, rows_v, acc, sem):
        c = lax.axis_index("c")
        s = lax.axis_index("s")
        tid = c * 16 + s

        pltpu.sync_copy(srcp_hbm.at[pl.ds(tid * CHUNKS, CHUNKS)], src_v)
        pltpu.sync_copy(dstp_hbm.at[pl.ds(tid * CHUNKS, CHUNKS)], dst_v)

        pltpu.sync_copy(zeros_hbm, acc.at[pl.ds(s * ZROWS, ZROWS)])
        plsc.subcore_barrier()

        def chunk_body(ci, carry):
            pltpu.async_copy(x_hbm.at[src_v.at[ci]], rows_v, sem).wait()
            pltpu.sync_copy(rows_v, acc.at[dst_v.at[ci]], add=True)
            return carry

        lax.fori_loop(0, CHUNKS, chunk_body, 0)
        plsc.subcore_barrier()

        pltpu.sync_copy(acc.at[pl.ds(s * ZROWS, ZROWS)],
                        out_hbm.at[c, pl.ds(s * ZROWS, ZROWS)])

    return _sc_aggregate


ROWS1 = 1000


def _mlp1_body(eps_ref, x_ref, p0_ref, p1_ref, w1_ref, b1_ref, h_ref, st_ref):
    i = pl.program_id(0)
    a = x_ref[...] * (1.0 + eps_ref[0, 0]) + p0_ref[...] + p1_ref[...]
    h = jnp.dot(a, w1_ref[...], preferred_element_type=jnp.float32) + b1_ref[...]
    h_ref[...] = h
    s = jnp.sum(h, axis=0, keepdims=True)
    s2 = jnp.sum(h * h, axis=0, keepdims=True)

    @pl.when(i == 0)
    def _():
        st_ref[0:1, :] = s
        st_ref[1:2, :] = s2

    @pl.when(i > 0)
    def _():
        st_ref[0:1, :] += s
        st_ref[1:2, :] += s2


_mlp1 = pl.pallas_call(
    _mlp1_body,
    grid=(N // ROWS1,),
    in_specs=[
        pl.BlockSpec(memory_space=pltpu.SMEM),
        pl.BlockSpec((ROWS1, D), lambda i: (i, 0)),
        pl.BlockSpec((ROWS1, D), lambda i: (i, 0)),
        pl.BlockSpec((ROWS1, D), lambda i: (i, 0)),
        pl.BlockSpec((D, D), lambda i: (0, 0)),
        pl.BlockSpec((1, D), lambda i: (0, 0)),
    ],
    out_specs=[
        pl.BlockSpec((ROWS1, D), lambda i: (i, 0)),
        pl.BlockSpec((2, D), lambda i: (0, 0)),
    ],
    out_shape=[
        jax.ShapeDtypeStruct((N, D), jnp.float32),
        jax.ShapeDtypeStruct((2, D), jnp.float32),
    ],
    compiler_params=pltpu.CompilerParams(dimension_semantics=("arbitrary",)),
)


def _mlp2_body(h_ref, st_ref, bng_ref, bnb_ref, w2_ref, b2_ref, lng_ref,
               lnb_ref, out_ref):
    m = st_ref[0:1, :] / N
    v = st_ref[1:2, :] / N - m * m
    h = (h_ref[...] - m) * lax.rsqrt(v + 1e-5) * bng_ref[...] + bnb_ref[...]
    h = jnp.maximum(h, 0.0)
    h = jnp.dot(h, w2_ref[...], preferred_element_type=jnp.float32) + b2_ref[...]
    mu = jnp.mean(h, axis=1, keepdims=True)
    va = jnp.mean(h * h, axis=1, keepdims=True) - mu * mu
    h = (h - mu) * lax.rsqrt(va + 1e-5) * lng_ref[...] + lnb_ref[...]
    out_ref[...] = jnp.where(h > 0, h, 0.1 * h)


_mlp2 = pl.pallas_call(
    _mlp2_body,
    grid=(N // ROWS1,),
    in_specs=[
        pl.BlockSpec((ROWS1, D), lambda i: (i, 0)),
        pl.BlockSpec((2, D), lambda i: (0, 0)),
        pl.BlockSpec((1, D), lambda i: (0, 0)),
        pl.BlockSpec((1, D), lambda i: (0, 0)),
        pl.BlockSpec((D, D), lambda i: (0, 0)),
        pl.BlockSpec((1, D), lambda i: (0, 0)),
        pl.BlockSpec((1, D), lambda i: (0, 0)),
        pl.BlockSpec((1, D), lambda i: (0, 0)),
    ],
    out_specs=pl.BlockSpec((ROWS1, D), lambda i: (i, 0)),
    out_shape=jax.ShapeDtypeStruct((N, D), jnp.float32),
    compiler_params=pltpu.CompilerParams(dimension_semantics=("arbitrary",)),
)


ROWSP = 200


def _pool_body(x_ref, b_ref, wf_ref, bf_ref, out_ref, mx, sm, cnt):
    i = pl.program_id(0)

    @pl.when(i == 0)
    def _():
        mx[...] = jnp.full((G, D), -jnp.inf, jnp.float32)
        sm[...] = jnp.zeros((G, D), jnp.float32)
        cnt[...] = jnp.zeros((G, 1), jnp.float32)

    x = x_ref[...]
    gids = lax.broadcasted_iota(jnp.int32, (1, G), 1)
    oh = (b_ref[...] == gids).astype(jnp.float32)
    dn = (((0,), (0,)), ((), ()))
    sm[...] += lax.dot_general(oh, x, dn, preferred_element_type=jnp.float32)
    cnt[...] += lax.dot_general(oh, jnp.ones((ROWSP, 1), jnp.float32), dn,
                                preferred_element_type=jnp.float32)
    b3 = lax.broadcast_in_dim(b_ref[...], (ROWSP, G, D), (0, 1))
    g3 = lax.broadcasted_iota(jnp.int32, (ROWSP, G, D), 1)
    x3 = lax.broadcast_in_dim(x, (ROWSP, G, D), (0, 2))
    big = jnp.where(b3 == g3, x3, -jnp.inf)
    mx[...] = jnp.maximum(mx[...], jnp.max(big, axis=0))

    @pl.when(i == pl.num_programs(0) - 1)
    def _():
        mean = sm[...] / jnp.maximum(cnt[...], 1.0)
        res = jnp.dot(mx[...], wf_ref[0:D, :], preferred_element_type=jnp.float32)
        res += jnp.dot(mean, wf_ref[D:2 * D, :], preferred_element_type=jnp.float32)
        out_ref[...] = res + bf_ref[...]


_pool = pl.pallas_call(
    _pool_body,
    grid=(N // ROWSP,),
    in_specs=[
        pl.BlockSpec((ROWSP, D), lambda i: (i, 0)),
        pl.BlockSpec((ROWSP, 1), lambda i: (i, 0)),
        pl.BlockSpec((2 * D, D), lambda i: (0, 0)),
        pl.BlockSpec((1, D), lambda i: (0, 0)),
    ],
    out_specs=pl.BlockSpec((G, D), lambda i: (0, 0)),
    out_shape=jax.ShapeDtypeStruct((G, D), jnp.float32),
    scratch_shapes=[
        pltpu.VMEM((G, D), jnp.float32),
        pltpu.VMEM((G, D), jnp.float32),
        pltpu.VMEM((G, 1), jnp.float32),
    ],
    compiler_params=pltpu.CompilerParams(dimension_semantics=("arbitrary",)),
)


def kernel(x, edge_index, batch, W1, b1, bn_g, bn_b, W2, b2, eps, ln_g, ln_b,
           Wf, bf):
    E = edge_index.shape[1]
    pad = EP - E
    src = jnp.concatenate([edge_index[0], jnp.zeros((pad,), jnp.int32)])
    dst = jnp.concatenate([edge_index[1], jnp.full((pad,), N, jnp.int32)])
    srcp = src.reshape(EP // CH, CH)
    dstp = dst.reshape(EP // CH, CH)
    zeros_blk = jnp.zeros((ZROWS, D), jnp.float32)
    batch2 = batch.reshape(N, 1)

    out = x
    for l in range(W1.shape[0]):
        partials = _get_sc_aggregate()(out, srcp, dstp, zeros_blk)
        h1, st = _mlp1(eps[l].reshape(1, 1), out, partials[0], partials[1],
                       W1[l], b1[l].reshape(1, D))
        out = _mlp2(h1, st, bn_g[l].reshape(1, D), bn_b[l].reshape(1, D),
                    W2[l], b2[l].reshape(1, D), ln_g[l].reshape(1, D),
                    ln_b[l].reshape(1, D))
    return _pool(out, batch2, Wf, bf.reshape(1, D))


# speedup vs baseline: 3.0211x; 1.1073x over previous
import functools

import jax
import jax.numpy as jnp
from jax import lax
from jax.experimental import pallas as pl
from jax.experimental.pallas import tpu as pltpu
from jax.experimental.pallas import tpu_sc as plsc

N = 10000
D = 128
G = 64

NTILES = 32
CH = 128
CHUNKS = 80
PH = 40
EPT = CH * CHUNKS
EP = NTILES * EPT
NACC = 10240
ZROWS = 640
NBUF = 2


@functools.cache
def _get_sc_aggregate():
    mesh = plsc.VectorSubcoreMesh(core_axis_name="c", subcore_axis_name="s")

    @functools.partial(
        pl.kernel,
        out_type=jax.ShapeDtypeStruct((2, NACC, D), jnp.float32),
        scratch_types=[
            pltpu.VMEM((PH, CH), jnp.int32),
            pltpu.VMEM((PH, CH), jnp.int32),
            pltpu.VMEM((NBUF, CH, D), jnp.float32),
            pltpu.VMEM_SHARED((NACC, D), jnp.float32),
        ] + [pltpu.SemaphoreType.DMA] * NBUF,
        mesh=mesh,
    )
    def _sc_aggregate(x_hbm, srcp_hbm, dstp_hbm, zeros_hbm, out_hbm,
                      src_v, dst_v, rows_v, acc, *sems):
        c = lax.axis_index("c")
        s = lax.axis_index("s")
        tid = c * 16 + s

        pltpu.sync_copy(zeros_hbm, acc.at[pl.ds(s * ZROWS, ZROWS)])
        plsc.subcore_barrier()

        for p in range(CHUNKS // PH):
            base = tid * CHUNKS + p * PH
            pltpu.sync_copy(srcp_hbm.at[pl.ds(base, PH)], src_v)
            pltpu.sync_copy(dstp_hbm.at[pl.ds(base, PH)], dst_v)

            for b in range(NBUF):
                pltpu.async_copy(x_hbm.at[src_v.at[b]], rows_v.at[b], sems[b])

            def ring_body(t, carry):
                for b in range(NBUF):
                    ci = t * NBUF + b
                    pltpu.make_async_copy(x_hbm.at[src_v.at[ci]],
                                          rows_v.at[b], sems[b]).wait()
                    pltpu.sync_copy(rows_v.at[b], acc.at[dst_v.at[ci]],
                                    add=True)
                    pltpu.async_copy(x_hbm.at[src_v.at[ci + NBUF]],
                                     rows_v.at[b], sems[b])
                return carry

            lax.fori_loop(0, PH // NBUF - 1, ring_body, 0)
            for b in range(NBUF):
                ci = PH - NBUF + b
                pltpu.make_async_copy(x_hbm.at[src_v.at[ci]], rows_v.at[b],
                                      sems[b]).wait()
                pltpu.sync_copy(rows_v.at[b], acc.at[dst_v.at[ci]], add=True)
        plsc.subcore_barrier()

        pltpu.sync_copy(acc.at[pl.ds(s * ZROWS, ZROWS)],
                        out_hbm.at[c, pl.ds(s * ZROWS, ZROWS)])

    return _sc_aggregate


ROWS1 = 1000


def _mlp1_body(eps_ref, x_ref, p0_ref, p1_ref, w1_ref, b1_ref, h_ref, st_ref):
    i = pl.program_id(0)
    a = x_ref[...] * (1.0 + eps_ref[0, 0]) + p0_ref[...] + p1_ref[...]
    h = jnp.dot(a, w1_ref[...], preferred_element_type=jnp.float32) + b1_ref[...]
    h_ref[...] = h
    s = jnp.sum(h, axis=0, keepdims=True)
    s2 = jnp.sum(h * h, axis=0, keepdims=True)

    @pl.when(i == 0)
    def _():
        st_ref[0:1, :] = s
        st_ref[1:2, :] = s2

    @pl.when(i > 0)
    def _():
        st_ref[0:1, :] += s
        st_ref[1:2, :] += s2


_mlp1 = pl.pallas_call(
    _mlp1_body,
    grid=(N // ROWS1,),
    in_specs=[
        pl.BlockSpec(memory_space=pltpu.SMEM),
        pl.BlockSpec((ROWS1, D), lambda i: (i, 0)),
        pl.BlockSpec((ROWS1, D), lambda i: (i, 0)),
        pl.BlockSpec((ROWS1, D), lambda i: (i, 0)),
        pl.BlockSpec((D, D), lambda i: (0, 0)),
        pl.BlockSpec((1, D), lambda i: (0, 0)),
    ],
    out_specs=[
        pl.BlockSpec((ROWS1, D), lambda i: (i, 0)),
        pl.BlockSpec((2, D), lambda i: (0, 0)),
    ],
    out_shape=[
        jax.ShapeDtypeStruct((N, D), jnp.float32),
        jax.ShapeDtypeStruct((2, D), jnp.float32),
    ],
    compiler_params=pltpu.CompilerParams(dimension_semantics=("arbitrary",)),
)


def _mlp2_body(h_ref, st_ref, bng_ref, bnb_ref, w2_ref, b2_ref, lng_ref,
               lnb_ref, out_ref):
    m = st_ref[0:1, :] / N
    v = st_ref[1:2, :] / N - m * m
    h = (h_ref[...] - m) * lax.rsqrt(v + 1e-5) * bng_ref[...] + bnb_ref[...]
    h = jnp.maximum(h, 0.0)
    h = jnp.dot(h, w2_ref[...], preferred_element_type=jnp.float32) + b2_ref[...]
    mu = jnp.mean(h, axis=1, keepdims=True)
    va = jnp.mean(h * h, axis=1, keepdims=True) - mu * mu
    h = (h - mu) * lax.rsqrt(va + 1e-5) * lng_ref[...] + lnb_ref[...]
    out_ref[...] = jnp.where(h > 0, h, 0.1 * h)


_mlp2 = pl.pallas_call(
    _mlp2_body,
    grid=(N // ROWS1,),
    in_specs=[
        pl.BlockSpec((ROWS1, D), lambda i: (i, 0)),
        pl.BlockSpec((2, D), lambda i: (0, 0)),
        pl.BlockSpec((1, D), lambda i: (0, 0)),
        pl.BlockSpec((1, D), lambda i: (0, 0)),
        pl.BlockSpec((D, D), lambda i: (0, 0)),
        pl.BlockSpec((1, D), lambda i: (0, 0)),
        pl.BlockSpec((1, D), lambda i: (0, 0)),
        pl.BlockSpec((1, D), lambda i: (0, 0)),
    ],
    out_specs=pl.BlockSpec((ROWS1, D), lambda i: (i, 0)),
    out_shape=jax.ShapeDtypeStruct((N, D), jnp.float32),
    compiler_params=pltpu.CompilerParams(dimension_semantics=("arbitrary",)),
)


ROWSP = 200


def _pool_body(x_ref, b_ref, wf_ref, bf_ref, out_ref, mx, sm, cnt):
    i = pl.program_id(0)

    @pl.when(i == 0)
    def _():
        mx[...] = jnp.full((G, D), -jnp.inf, jnp.float32)
        sm[...] = jnp.zeros((G, D), jnp.float32)
        cnt[...] = jnp.zeros((G, 1), jnp.float32)

    x = x_ref[...]
    gids = lax.broadcasted_iota(jnp.int32, (1, G), 1)
    oh = (b_ref[...] == gids).astype(jnp.float32)
    dn = (((0,), (0,)), ((), ()))
    sm[...] += lax.dot_general(oh, x, dn, preferred_element_type=jnp.float32)
    cnt[...] += lax.dot_general(oh, jnp.ones((ROWSP, 1), jnp.float32), dn,
                                preferred_element_type=jnp.float32)
    b3 = lax.broadcast_in_dim(b_ref[...], (ROWSP, G, D), (0, 1))
    g3 = lax.broadcasted_iota(jnp.int32, (ROWSP, G, D), 1)
    x3 = lax.broadcast_in_dim(x, (ROWSP, G, D), (0, 2))
    big = jnp.where(b3 == g3, x3, -jnp.inf)
    mx[...] = jnp.maximum(mx[...], jnp.max(big, axis=0))

    @pl.when(i == pl.num_programs(0) - 1)
    def _():
        mean = sm[...] / jnp.maximum(cnt[...], 1.0)
        res = jnp.dot(mx[...], wf_ref[0:D, :], preferred_element_type=jnp.float32)
        res += jnp.dot(mean, wf_ref[D:2 * D, :], preferred_element_type=jnp.float32)
        out_ref[...] = res + bf_ref[...]


_pool = pl.pallas_call(
    _pool_body,
    grid=(N // ROWSP,),
    in_specs=[
        pl.BlockSpec((ROWSP, D), lambda i: (i, 0)),
        pl.BlockSpec((ROWSP, 1), lambda i: (i, 0)),
        pl.BlockSpec((2 * D, D), lambda i: (0, 0)),
        pl.BlockSpec((1, D), lambda i: (0, 0)),
    ],
    out_specs=pl.BlockSpec((G, D), lambda i: (0, 0)),
    out_shape=jax.ShapeDtypeStruct((G, D), jnp.float32),
    scratch_shapes=[
        pltpu.VMEM((G, D), jnp.float32),
        pltpu.VMEM((G, D), jnp.float32),
        pltpu.VMEM((G, 1), jnp.float32),
    ],
    compiler_params=pltpu.CompilerParams(dimension_semantics=("arbitrary",)),
)


def kernel(x, edge_index, batch, W1, b1, bn_g, bn_b, W2, b2, eps, ln_g, ln_b,
           Wf, bf):
    E = edge_index.shape[1]
    pad = EP - E
    src = jnp.concatenate([edge_index[0], jnp.zeros((pad,), jnp.int32)])
    dst = jnp.concatenate([edge_index[1], jnp.full((pad,), N, jnp.int32)])
    srcp = src.reshape(EP // CH, CH)
    dstp = dst.reshape(EP // CH, CH)
    zeros_blk = jnp.zeros((ZROWS, D), jnp.float32)
    batch2 = batch.reshape(N, 1)

    out = x
    for l in range(W1.shape[0]):
        partials = _get_sc_aggregate()(out, srcp, dstp, zeros_blk)
        h1, st = _mlp1(eps[l].reshape(1, 1), out, partials[0], partials[1],
                       W1[l], b1[l].reshape(1, D))
        out = _mlp2(h1, st, bn_g[l].reshape(1, D), bn_b[l].reshape(1, D),
                    W2[l], b2[l].reshape(1, D), ln_g[l].reshape(1, D),
                    ln_b[l].reshape(1, D))
    return _pool(out, batch2, Wf, bf.reshape(1, D))
